# Initial kernel scaffold; baseline (speedup 1.0000x reference)
import jax
import jax.numpy as jnp
from jax.experimental import pallas as pl


def kernel(x, enc_W0, enc_b0, enc_W1, enc_b1, enc_W2, enc_b2, enc_W3, enc_b3, dec_W0, dec_b0, dec_W1, dec_b1, dec_W2, dec_b2, dec_W3, dec_b3, ln_g, ln_b, codebooks):
    raise NotImplementedError("write your pallas kernel here")


# speedup vs baseline: 1.3433x; 1.3433x over previous
import jax
import jax.numpy as jnp
from jax.experimental import pallas as pl

_B = 16384
_IN = 768
_HID = 32
_K = 256
_L = 3
_BETA = 0.25
_TB = 512


def _fused(x_ref,
           ew0, eb0, ew1, eb1, ew2, eb2, ew3, eb3,
           dw0, db0, dw1, db1, dw2, db2, dw3, db3,
           lng, lnb, cb_ref,
           out_ref, idx_ref, qrep_ref, loss_ref):
    f32 = jnp.float32
    bf16 = jnp.bfloat16

    def mm(a, b):
        return jnp.dot(a.astype(bf16), b.astype(bf16),
                       preferred_element_type=f32)

    h = x_ref[...]
    for w, b, act in ((ew0, eb0, True), (ew1, eb1, True),
                      (ew2, eb2, True), (ew3, eb3, False)):
        h = mm(h, w[...]) + b[...]
        if act:
            h = jnp.maximum(h, 0.0)
    mu = jnp.mean(h, axis=1, keepdims=True)
    var = jnp.mean((h - mu) ** 2, axis=1, keepdims=True)
    enc = (h - mu) / jnp.sqrt(var + 1e-5) * lng[...] + lnb[...]
    res = enc
    qrep = jnp.zeros_like(enc)
    sumsq = jnp.asarray(0.0, f32)
    idx_cols = []
    iota = jax.lax.broadcasted_iota(jnp.int32, (_TB, _K), 1)
    ones_row = jnp.ones((1, _HID), f32)
    for level in range(_L):
        cb = cb_ref[level]
        x2 = jnp.sum(res * res, axis=1, keepdims=True)
        y2 = jax.lax.dot_general(ones_row, cb * cb,
                                 (((1,), (1,)), ((), ())),
                                 preferred_element_type=f32,
                                 precision=jax.lax.Precision.HIGHEST)
        rc = jax.lax.dot_general(res.astype(jnp.bfloat16),
                                 cb.astype(jnp.bfloat16),
                                 (((1,), (1,)), ((), ())),
                                 preferred_element_type=f32)
        d = (x2 + y2) - 2.0 * rc
        m = jnp.min(d, axis=1, keepdims=True)
        idx = jnp.min(jnp.where(d <= m, iota, _K), axis=1, keepdims=True)
        oh = (iota == idx).astype(f32)
        qv = jnp.dot(oh, cb, preferred_element_type=f32,
                     precision=jax.lax.Precision.HIGHEST)
        sumsq = sumsq + jnp.sum((res - qv) ** 2)
        qrep = qrep + qv
        res = res - qv
        idx_cols.append(idx)
    qrep_ref[...] = qrep
    idx_ref[...] = jnp.concatenate(idx_cols, axis=1)
    h = qrep
    for w, b, act in ((dw0, db0, True), (dw1, db1, True),
                      (dw2, db2, True), (dw3, db3, False)):
        h = mm(h, w[...]) + b[...]
        if act:
            h = jnp.maximum(h, 0.0)
    out_ref[...] = h
    part = jnp.reshape(_BETA * sumsq / jnp.asarray(_B * _HID, f32), (1, 1))
    @pl.when(pl.program_id(0) == 0)
    def _init():
        loss_ref[...] = part
    @pl.when(pl.program_id(0) != 0)
    def _acc():
        loss_ref[...] = loss_ref[...] + part


def kernel(x, enc_W0, enc_b0, enc_W1, enc_b1, enc_W2, enc_b2, enc_W3, enc_b3,
           dec_W0, dec_b0, dec_W1, dec_b1, dec_W2, dec_b2, dec_W3, dec_b3,
           ln_g, ln_b, codebooks):
    eb = [b.reshape(1, -1) for b in (enc_b0, enc_b1, enc_b2, enc_b3)]
    db = [b.reshape(1, -1) for b in (dec_b0, dec_b1, dec_b2, dec_b3)]
    lng = ln_g.reshape(1, -1)
    lnb = ln_b.reshape(1, -1)

    def fixed(shape):
        return pl.BlockSpec(shape, lambda i: (0,) * len(shape))

    in_specs = [pl.BlockSpec((_TB, _IN), lambda i: (i, 0))]
    ops = []
    enc_ws = (enc_W0, enc_W1, enc_W2, enc_W3)
    dec_ws = (dec_W0, dec_W1, dec_W2, dec_W3)
    for w, b in zip(enc_ws, eb):
        in_specs += [fixed(w.shape), fixed(b.shape)]
        ops += [w, b]
    for w, b in zip(dec_ws, db):
        in_specs += [fixed(w.shape), fixed(b.shape)]
        ops += [w, b]
    in_specs += [fixed(lng.shape), fixed(lnb.shape), fixed(codebooks.shape)]
    ops += [lng, lnb, codebooks]

    out_shape = [
        jax.ShapeDtypeStruct((_B, _IN), jnp.float32),
        jax.ShapeDtypeStruct((_B, _L), jnp.int32),
        jax.ShapeDtypeStruct((_B, _HID), jnp.float32),
        jax.ShapeDtypeStruct((1, 1), jnp.float32),
    ]
    out_specs = [
        pl.BlockSpec((_TB, _IN), lambda i: (i, 0)),
        pl.BlockSpec((_TB, _L), lambda i: (i, 0)),
        pl.BlockSpec((_TB, _HID), lambda i: (i, 0)),
        pl.BlockSpec((1, 1), lambda i: (0, 0)),
    ]
    out, idx, qrep, loss = pl.pallas_call(
        _fused,
        grid=(_B // _TB,),
        in_specs=in_specs,
        out_specs=out_specs,
        out_shape=out_shape,
    )(x, *ops)
    return (out, idx, qrep, loss[0, 0])


# speedup vs baseline: 1.4335x; 1.0672x over previous
import jax
import jax.numpy as jnp
from jax.experimental import pallas as pl

_B = 16384
_IN = 768
_HID = 32
_K = 256
_L = 3
_BETA = 0.25
_TB = 1024


def _fused(x_ref,
           ew0, eb0, ew1, eb1, ew2, eb2, ew3, eb3,
           dw0, db0, dw1, db1, dw2, db2, dw3, db3,
           lng, lnb, cb_ref, cbh_ref,
           out_ref, idx_ref, qrep_ref, loss_ref):
    f32 = jnp.float32
    bf16 = jnp.bfloat16

    def mm(a, w):
        return jnp.dot(a.astype(bf16), w[...], preferred_element_type=f32)

    h = x_ref[...]
    for w, b, act in ((ew0, eb0, True), (ew1, eb1, True),
                      (ew2, eb2, True), (ew3, eb3, False)):
        h = mm(h, w) + b[...]
        if act:
            h = jnp.maximum(h, 0.0)
    mu = jnp.mean(h, axis=1, keepdims=True)
    var = jnp.mean((h - mu) ** 2, axis=1, keepdims=True)
    enc = (h - mu) / jnp.sqrt(var + 1e-5) * lng[...] + lnb[...]
    res = enc
    qrep = jnp.zeros_like(enc)
    sumsq = jnp.asarray(0.0, f32)
    idx_cols = []
    iota = jax.lax.broadcasted_iota(jnp.int32, (_TB, _K), 1)
    ones_row = jnp.ones((1, _HID), f32)
    for level in range(_L):
        cb = cb_ref[level]
        x2 = jnp.sum(res * res, axis=1, keepdims=True)
        y2 = jax.lax.dot_general(ones_row, cb * cb,
                                 (((1,), (1,)), ((), ())),
                                 preferred_element_type=f32,
                                 precision=jax.lax.Precision.HIGHEST)
        rc = jax.lax.dot_general(res.astype(bf16), cbh_ref[level],
                                 (((1,), (1,)), ((), ())),
                                 preferred_element_type=f32)
        d = (x2 + y2) - 2.0 * rc
        m = jnp.min(d, axis=1, keepdims=True)
        idx = jnp.min(jnp.where(d <= m, iota, _K), axis=1, keepdims=True)
        oh = (iota == idx).astype(f32)
        qv = jnp.dot(oh, cb, preferred_element_type=f32,
                     precision=jax.lax.Precision.HIGHEST)
        sumsq = sumsq + jnp.sum((res - qv) ** 2)
        qrep = qrep + qv
        res = res - qv
        idx_cols.append(idx)
    qrep_ref[...] = qrep
    idx_ref[...] = jnp.concatenate(idx_cols, axis=1)
    h = qrep
    for w, b, act in ((dw0, db0, True), (dw1, db1, True),
                      (dw2, db2, True), (dw3, db3, False)):
        h = mm(h, w) + b[...]
        if act:
            h = jnp.maximum(h, 0.0)
    out_ref[...] = h
    part = jnp.reshape(_BETA * sumsq / jnp.asarray(_B * _HID, jnp.float32),
                       (1, 1))
    @pl.when(pl.program_id(0) == 0)
    def _init():
        loss_ref[...] = part
    @pl.when(pl.program_id(0) != 0)
    def _acc():
        loss_ref[...] = loss_ref[...] + part


def kernel(x, enc_W0, enc_b0, enc_W1, enc_b1, enc_W2, enc_b2, enc_W3, enc_b3,
           dec_W0, dec_b0, dec_W1, dec_b1, dec_W2, dec_b2, dec_W3, dec_b3,
           ln_g, ln_b, codebooks):
    eb = [b.reshape(1, -1) for b in (enc_b0, enc_b1, enc_b2, enc_b3)]
    db = [b.reshape(1, -1) for b in (dec_b0, dec_b1, dec_b2, dec_b3)]
    lng = ln_g.reshape(1, -1)
    lnb = ln_b.reshape(1, -1)
    ews = [w.astype(jnp.bfloat16) for w in (enc_W0, enc_W1, enc_W2, enc_W3)]
    dws = [w.astype(jnp.bfloat16) for w in (dec_W0, dec_W1, dec_W2, dec_W3)]
    cb_bf16 = codebooks.astype(jnp.bfloat16)

    def fixed(shape):
        return pl.BlockSpec(shape, lambda i: (0,) * len(shape))

    in_specs = [pl.BlockSpec((_TB, _IN), lambda i: (i, 0))]
    ops = []
    for w, b in zip(ews, eb):
        in_specs += [fixed(w.shape), fixed(b.shape)]
        ops += [w, b]
    for w, b in zip(dws, db):
        in_specs += [fixed(w.shape), fixed(b.shape)]
        ops += [w, b]
    in_specs += [fixed(lng.shape), fixed(lnb.shape),
                 fixed(codebooks.shape), fixed(cb_bf16.shape)]
    ops += [lng, lnb, codebooks, cb_bf16]

    out_shape = [
        jax.ShapeDtypeStruct((_B, _IN), jnp.float32),
        jax.ShapeDtypeStruct((_B, _L), jnp.int32),
        jax.ShapeDtypeStruct((_B, _HID), jnp.float32),
        jax.ShapeDtypeStruct((1, 1), jnp.float32),
    ]
    out_specs = [
        pl.BlockSpec((_TB, _IN), lambda i: (i, 0)),
        pl.BlockSpec((_TB, _L), lambda i: (i, 0)),
        pl.BlockSpec((_TB, _HID), lambda i: (i, 0)),
        pl.BlockSpec((1, 1), lambda i: (0, 0)),
    ]
    out, idx, qrep, loss = pl.pallas_call(
        _fused,
        grid=(_B // _TB,),
        in_specs=in_specs,
        out_specs=out_specs,
        out_shape=out_shape,
    )(x, *ops)
    return (out, idx, qrep, loss[0, 0])
